# baseline (speedup 1.0000x reference)
import functools

import jax
import jax.numpy as jnp
from jax import lax
from jax.experimental import pallas as pl
from jax.experimental.pallas import tpu as pltpu
from jax.experimental.pallas import tpu_sc as plsc

L = 16

B = 16
N = 32768
D = 64
K = round(N * 0.15)

NVEC = N // L
CHUNK = 128
NCHUNK = -(-K // CHUNK)
KPAD = NCHUNK * CHUNK
LAST = K - (NCHUNK - 1) * CHUNK
HSTRIDE = 257


def _topk_body(frames_hbm, scores_hbm, out_hbm,
               scores_v, sv, hist, suffix, idx_v, gbuf, sem):
    c = lax.axis_index("c")
    s = lax.axis_index("s")
    batch = c * 8 + s

    @pl.when(s < 8)
    def _():
        lanes = lax.iota(jnp.int32, L)
        ones = jnp.ones((L,), jnp.int32)
        zeros = jnp.zeros((L,), jnp.int32)
        base0 = batch * N

        pltpu.sync_copy(scores_hbm.at[batch], scores_v)

        suffix[pl.ds(256, L)] = zeros

        def zero_hist(i, carry):
            hist[pl.ds(i * L, L)] = zeros
            return carry

        lax.fori_loop(0, HSTRIDE, zero_hist, 0)

        def cvt(i, carry):
            x = scores_v[pl.ds(i * L, L)]
            b = plsc.bitcast(x, jnp.int32)
            key = b ^ lax.shift_right_logical(b >> 31, 1)
            sv[pl.ds(i * L, L)] = key
            bin0 = (key >> 24) + 128
            plsc.addupdate_scatter(hist, [bin0 + lanes * HSTRIDE], ones)
            return carry

        lax.fori_loop(0, NVEC, cvt, 0)

        k_rem = jnp.int32(K)
        prefix = jnp.int32(0)

        for level in range(4):
            shift = 24 - 8 * level
            if level > 0:
                lax.fori_loop(0, HSTRIDE, zero_hist, 0)
                match_mask = jnp.int32(-(1 << (shift + 8)))
                pref = prefix

                def hpass(i, carry):
                    key = sv[pl.ds(i * L, L)]
                    filt = (key & match_mask) == pref
                    bins = (key >> shift) & 0xFF
                    plsc.addupdate_scatter(
                        hist, [bins + lanes * HSTRIDE], ones, mask=filt)
                    return carry

                lax.fori_loop(0, NVEC, hpass, 0)

            carry = jnp.int32(0)
            for j in reversed(range(L)):
                tot = hist[pl.ds(j * L, L)]
                for lane in range(1, L):
                    tot = tot + hist[pl.ds(lane * HSTRIDE + j * L, L)]
                cs = plsc.cumsum(lax.rev(tot, (0,))) + carry
                suffix[pl.ds(j * L, L)] = lax.rev(cs, (0,))
                carry = jnp.max(cs)

            acc = zeros
            for j in range(L):
                sfx = suffix[pl.ds(j * L, L)]
                acc = acc + jnp.where(sfx >= k_rem, 1, 0).astype(jnp.int32)
            bsel = jnp.sum(acc) - 1

            v1 = bsel + 1
            sfx = suffix[pl.ds((v1 >> 4) << 4, L)]
            c_above = jnp.sum(jnp.where(lanes == (v1 & 15), sfx, 0))
            k_rem = k_rem - c_above
            if level == 0:
                prefix = (bsel - 128) << 24
            else:
                prefix = prefix | (bsel << shift)

        thresh = prefix
        need = k_rem

        def prefill(i, carry):
            idx_v[pl.ds(i * L, L)] = base0 + i * L + lanes
            return carry

        lax.fori_loop(0, KPAD // L, prefill, 0)

        def sel(i, carry):
            n_sel, eq_seen = carry
            key = sv[pl.ds(i * L, L)]
            mgt = key > thresh
            meq = key == thresh
            csum = plsc.cumsum(jnp.where(meq, 1, 0).astype(jnp.int32))
            m = mgt | (meq & ((eq_seen + csum) <= need))
            vals = base0 + i * L + lanes
            plsc.store_compressed(idx_v.at[pl.ds(n_sel, L)], vals, mask=m)
            cnt = plsc.all_reduce_population_count(m)
            return n_sel + jnp.max(cnt), eq_seen + jnp.max(csum)

        lax.fori_loop(0, NVEC, sel, (jnp.int32(0), jnp.int32(0)))

        for chunk in range(NCHUNK):
            cp = pltpu.async_copy(
                frames_hbm.at[idx_v.at[pl.ds(chunk * CHUNK, CHUNK)]],
                gbuf, sem)
            cp.wait()
            if chunk < NCHUNK - 1:
                pltpu.sync_copy(
                    gbuf, out_hbm.at[batch].at[pl.ds(chunk * CHUNK, CHUNK)])
            else:
                pltpu.sync_copy(
                    gbuf.at[pl.ds(0, LAST)],
                    out_hbm.at[batch].at[pl.ds(chunk * CHUNK, LAST)])


@jax.jit
def _sc_topk(frames2d, scores):
    mesh = plsc.VectorSubcoreMesh(core_axis_name="c", subcore_axis_name="s")
    fn = functools.partial(
        pl.kernel,
        out_type=jax.ShapeDtypeStruct((B, K, D), jnp.float32),
        mesh=mesh,
        compiler_params=pltpu.CompilerParams(
            needs_layout_passes=False, use_tc_tiling_on_sc=False),
        scratch_types=[
            pltpu.VMEM((N,), jnp.float32),
            pltpu.VMEM((N,), jnp.int32),
            pltpu.VMEM((L * HSTRIDE,), jnp.int32),
            pltpu.VMEM((256 + L,), jnp.int32),
            pltpu.VMEM((KPAD,), jnp.int32),
            pltpu.VMEM((CHUNK, D), jnp.float32),
            pltpu.SemaphoreType.DMA,
        ],
    )(_topk_body)
    return fn(frames2d, scores)


def kernel(frames, scores):
    frames2d = frames.reshape(B * N, D)
    return _sc_topk(frames2d, scores)


# speedup vs baseline: 4.4079x; 4.4079x over previous
import functools

import jax
import jax.numpy as jnp
from jax import lax
from jax.experimental import pallas as pl
from jax.experimental.pallas import tpu as pltpu
from jax.experimental.pallas import tpu_sc as plsc

L = 16

B = 16
N = 32768
D = 64
K = round(N * 0.15)

NVEC = N // L
KPAD = 4992
NIDX = KPAD // L
HSTRIDE = 257
DHALF = D // 2


def _topk_body(fp2, scores_hbm, out_hbm,
               keys_v, row2_v, hist, suffix, idx_v, ob0, ob1, sp_idx,
               rs0, rs1, os0, os1):
    c = lax.axis_index("c")
    s = lax.axis_index("s")
    q = s >> 1
    h = s & 1
    batch = c * 8 + q

    lanes = lax.iota(jnp.int32, L)
    ones = jnp.ones((L,), jnp.int32)
    zeros = jnp.zeros((L,), jnp.int32)

    @pl.when(h == 0)
    def _():
        pltpu.sync_copy(scores_hbm.at[batch], keys_v)

        suffix[pl.ds(256, L)] = zeros

        @plsc.parallel_loop(0, HSTRIDE, unroll=8)
        def _(i):
            hist[pl.ds(i * L, L)] = zeros

        @plsc.parallel_loop(0, NVEC, unroll=8)
        def _(i):
            x = keys_v[pl.ds(i * L, L)]
            b = plsc.bitcast(x, jnp.int32)
            key = b ^ lax.shift_right_logical(b >> 31, 1)
            keys_v[pl.ds(i * L, L)] = plsc.bitcast(key, jnp.float32)
            bin0 = (key >> 24) + 128
            plsc.addupdate_scatter(hist, [bin0 + lanes * HSTRIDE], ones)

        k_rem = jnp.int32(K)
        prefix = jnp.int32(0)

        for level in range(4):
            shift = 24 - 8 * level
            if level > 0:
                match_mask = jnp.int32(-(1 << (shift + 8)))
                pref = prefix

                @plsc.parallel_loop(0, HSTRIDE, unroll=8)
                def _(i):
                    hist[pl.ds(i * L, L)] = zeros

                @plsc.parallel_loop(0, NVEC, unroll=8)
                def _(i):
                    key = plsc.bitcast(keys_v[pl.ds(i * L, L)], jnp.int32)
                    filt = (key & match_mask) == pref
                    bins = (key >> shift) & 0xFF
                    plsc.addupdate_scatter(
                        hist, [bins + lanes * HSTRIDE], ones, mask=filt)

            carry = jnp.int32(0)
            for j in reversed(range(L)):
                tot = hist[pl.ds(j * L, L)]
                for lane in range(1, L):
                    tot = tot + hist[pl.ds(lane * HSTRIDE + j * L, L)]
                cs = plsc.cumsum(lax.rev(tot, (0,))) + carry
                suffix[pl.ds(j * L, L)] = lax.rev(cs, (0,))
                carry = jnp.max(cs)

            acc = zeros
            for j in range(L):
                sfx = suffix[pl.ds(j * L, L)]
                acc = acc + jnp.where(sfx >= k_rem, 1, 0).astype(jnp.int32)
            bsel = jnp.sum(acc) - 1

            v1 = bsel + 1
            sfx = suffix[pl.ds((v1 >> 4) << 4, L)]
            c_above = jnp.sum(jnp.where(lanes == (v1 & 15), sfx, 0))
            k_rem = k_rem - c_above
            if level == 0:
                prefix = (bsel - 128) << 24
            else:
                prefix = prefix | (bsel << shift)

        thresh = prefix
        need = k_rem

        @plsc.parallel_loop(0, NIDX, unroll=8)
        def _(i):
            idx_v[pl.ds(i * L, L)] = i * L + lanes

        def sel(i, carry_in):
            n_sel, eq_seen = carry_in
            key = plsc.bitcast(keys_v[pl.ds(i * L, L)], jnp.int32)
            mgt = key > thresh
            meq = key == thresh
            csum = plsc.cumsum(jnp.where(meq, 1, 0).astype(jnp.int32))
            m = mgt | (meq & ((eq_seen + csum) <= need))
            plsc.store_compressed(idx_v.at[pl.ds(n_sel, L)], i * L + lanes,
                                  mask=m)
            cnt = plsc.all_reduce_population_count(m)
            return n_sel + jnp.max(cnt), eq_seen + jnp.max(csum)

        lax.fori_loop(0, NVEC, sel, (jnp.int32(0), jnp.int32(0)))

        pltpu.sync_copy(idx_v, sp_idx.at[q])

    plsc.subcore_barrier()

    @pl.when(h == 1)
    def _():
        pltpu.sync_copy(sp_idx.at[q], idx_v)

    rbase = batch * D + h * DHALF
    rbufs = (keys_v, row2_v)
    rsems = (rs0, rs1)
    obufs = (ob0, ob1)
    osems = (os0, os1)

    row_cp = [None, None]
    out_cp = [None, None]
    row_cp[0] = pltpu.async_copy(fp2.at[rbase], rbufs[0], rsems[0])
    for di in range(DHALF):
        p = di & 1
        nx = (di + 1) & 1
        if di + 1 < DHALF:
            row_cp[nx] = pltpu.async_copy(
                fp2.at[rbase + di + 1], rbufs[nx], rsems[nx])
        row_cp[p].wait()
        if out_cp[p] is not None:
            out_cp[p].wait()
        rb = rbufs[p]
        ob = obufs[p]

        @plsc.parallel_loop(0, NIDX, unroll=8)
        def _(i):
            iv = idx_v[pl.ds(i * L, L)]
            ob[pl.ds(i * L, L)] = plsc.load_gather(rb, [iv])

        out_cp[p] = pltpu.async_copy(ob, out_hbm.at[rbase + di], osems[p])
    out_cp[0].wait()
    out_cp[1].wait()


@jax.jit
def _sc_topk(fp2, scores):
    mesh = plsc.VectorSubcoreMesh(core_axis_name="c", subcore_axis_name="s")
    fn = functools.partial(
        pl.kernel,
        out_type=jax.ShapeDtypeStruct((B * D, KPAD), jnp.float32),
        mesh=mesh,
        compiler_params=pltpu.CompilerParams(
            needs_layout_passes=False, use_tc_tiling_on_sc=True),
        scratch_types=[
            pltpu.VMEM((N,), jnp.float32),
            pltpu.VMEM((N,), jnp.float32),
            pltpu.VMEM((L * HSTRIDE,), jnp.int32),
            pltpu.VMEM((256 + L,), jnp.int32),
            pltpu.VMEM((KPAD,), jnp.int32),
            pltpu.VMEM((KPAD,), jnp.float32),
            pltpu.VMEM((KPAD,), jnp.float32),
            pltpu.VMEM_SHARED((8, KPAD), jnp.int32),
            pltpu.SemaphoreType.DMA,
            pltpu.SemaphoreType.DMA,
            pltpu.SemaphoreType.DMA,
            pltpu.SemaphoreType.DMA,
        ],
    )(_topk_body)
    return fn(fp2, scores)


def kernel(frames, scores):
    fp2 = frames.transpose(0, 2, 1).reshape(B * D, N)
    out = _sc_topk(fp2, scores)
    return out[:, :K].reshape(B, D, K).transpose(0, 2, 1)


# speedup vs baseline: 5.1432x; 1.1668x over previous
import functools

import jax
import jax.numpy as jnp
from jax import lax
from jax.experimental import pallas as pl
from jax.experimental.pallas import tpu as pltpu
from jax.experimental.pallas import tpu_sc as plsc

L = 16

B = 16
N = 32768
D = 64
K = round(N * 0.15)

NVEC = N // L
KPAD = 4992
NIDX = KPAD // L
HSTRIDE = 257
DHALF = D // 2


def _topk_body(fp2, scores_hbm, out_hbm,
               keys_v, row2_v, hist, suffix, idx_v, ob0, ob1,
               cgt_v, ceq_v, off_v, eqp_v, sp_idx,
               rs0, rs1, os0, os1):
    c = lax.axis_index("c")
    s = lax.axis_index("s")
    q = s >> 1
    h = s & 1
    batch = c * 8 + q

    lanes = lax.iota(jnp.int32, L)
    ones = jnp.ones((L,), jnp.int32)
    zeros = jnp.zeros((L,), jnp.int32)

    rbase = batch * D + h * DHALF

    pltpu.async_copy(fp2.at[rbase + 1], row2_v, rs1)

    @pl.when(h == 1)
    def _():
        pltpu.async_copy(fp2.at[rbase], keys_v, rs0)

    @pl.when(h == 0)
    def _():
        pltpu.sync_copy(scores_hbm.at[batch], keys_v)

        suffix[pl.ds(256, L)] = zeros

        @plsc.parallel_loop(0, HSTRIDE, unroll=8)
        def _(i):
            hist[pl.ds(i * L, L)] = zeros

        @plsc.parallel_loop(0, NVEC, unroll=8)
        def _(i):
            x = keys_v[pl.ds(i * L, L)]
            b = plsc.bitcast(x, jnp.int32)
            key = b ^ lax.shift_right_logical(b >> 31, 1)
            keys_v[pl.ds(i * L, L)] = plsc.bitcast(key, jnp.float32)
            bin0 = (key >> 24) + 128
            plsc.addupdate_scatter(hist, [bin0 + lanes * HSTRIDE], ones)

        k_rem = jnp.int32(K)
        prefix = jnp.int32(0)

        for level in range(4):
            shift = 24 - 8 * level
            if level > 0:
                match_mask = jnp.int32(-(1 << (shift + 8)))
                pref = prefix

                @plsc.parallel_loop(0, HSTRIDE, unroll=8)
                def _(i):
                    hist[pl.ds(i * L, L)] = zeros

                @plsc.parallel_loop(0, NVEC, unroll=8)
                def _(i):
                    key = plsc.bitcast(keys_v[pl.ds(i * L, L)], jnp.int32)
                    filt = (key & match_mask) == pref
                    bins = (key >> shift) & 0xFF
                    plsc.addupdate_scatter(
                        hist, [bins + lanes * HSTRIDE], ones, mask=filt)

            carry = jnp.int32(0)
            for j in reversed(range(L)):
                tot = hist[pl.ds(j * L, L)]
                for lane in range(1, L):
                    tot = tot + hist[pl.ds(lane * HSTRIDE + j * L, L)]
                cs = plsc.cumsum(lax.rev(tot, (0,))) + carry
                suffix[pl.ds(j * L, L)] = lax.rev(cs, (0,))
                carry = jnp.max(cs)

            acc = zeros
            for j in range(L):
                sfx = suffix[pl.ds(j * L, L)]
                acc = acc + jnp.where(sfx >= k_rem, 1, 0).astype(jnp.int32)
            bsel = jnp.sum(acc) - 1

            v1 = bsel + 1
            sfx = suffix[pl.ds((v1 >> 4) << 4, L)]
            c_above = jnp.sum(jnp.where(lanes == (v1 & 15), sfx, 0))
            k_rem = k_rem - c_above
            if level == 0:
                prefix = (bsel - 128) << 24
            else:
                prefix = prefix | (bsel << shift)

        thresh = prefix
        need = k_rem

        @plsc.parallel_loop(0, NIDX, unroll=8)
        def _(i):
            idx_v[pl.ds(i * L, L)] = i * L + lanes

        lane0 = lanes == 0

        @plsc.parallel_loop(0, NVEC, unroll=8)
        def _(i):
            key = plsc.bitcast(keys_v[pl.ds(i * L, L)], jnp.int32)
            cgt = plsc.all_reduce_population_count(key > thresh)
            ceq = plsc.all_reduce_population_count(key == thresh)
            plsc.store_compressed(cgt_v.at[pl.ds(i, L)], cgt, mask=lane0)
            plsc.store_compressed(ceq_v.at[pl.ds(i, L)], ceq, mask=lane0)

        def scan(j, carry_in):
            n_off, eq_off = carry_in
            cgt = cgt_v[pl.ds(j * L, L)]
            ceq = ceq_v[pl.ds(j * L, L)]
            eq_incl = plsc.cumsum(ceq) + eq_off
            eq_excl = eq_incl - ceq
            sel_eq = jnp.minimum(jnp.maximum(need - eq_excl, 0), ceq)
            tot = cgt + sel_eq
            off_incl = plsc.cumsum(tot) + n_off
            off_v[pl.ds(j * L, L)] = off_incl - tot
            eqp_v[pl.ds(j * L, L)] = eq_excl
            return jnp.max(off_incl), jnp.max(eq_incl)

        lax.fori_loop(0, NVEC // L, scan, (jnp.int32(0), jnp.int32(0)))

        @plsc.parallel_loop(0, NVEC, unroll=8)
        def _(i):
            key = plsc.bitcast(keys_v[pl.ds(i * L, L)], jnp.int32)
            mgt = key > thresh
            meq = key == thresh
            csum = plsc.cumsum(jnp.where(meq, 1, 0).astype(jnp.int32))
            eqpre = eqp_v[pl.ds(i, L)][0]
            off = off_v[pl.ds(i, L)][0]
            m = mgt | (meq & ((eqpre + csum) <= need))
            plsc.store_compressed(idx_v.at[pl.ds(off, L)],
                                  i * L + lanes, mask=m)

        pltpu.async_copy(fp2.at[rbase], keys_v, rs0)

        pltpu.sync_copy(idx_v, sp_idx.at[q])

    plsc.subcore_barrier()

    @pl.when(h == 1)
    def _():
        pltpu.sync_copy(sp_idx.at[q], idx_v)

    rbufs = (keys_v, row2_v)
    rsems = (rs0, rs1)
    obufs = (ob0, ob1)
    osems = (os0, os1)

    row_cp = [
        pltpu.make_async_copy(fp2.at[rbase], rbufs[0], rsems[0]),
        pltpu.make_async_copy(fp2.at[rbase + 1], rbufs[1], rsems[1]),
    ]
    out_cp = [None, None]
    for di in range(DHALF):
        p = di & 1
        nx = (di + 1) & 1
        if 2 <= di + 1 < DHALF:
            row_cp[nx] = pltpu.async_copy(
                fp2.at[rbase + di + 1], rbufs[nx], rsems[nx])
        row_cp[p].wait()
        if out_cp[p] is not None:
            out_cp[p].wait()
        rb = rbufs[p]
        ob = obufs[p]

        @plsc.parallel_loop(0, NIDX, unroll=8)
        def _(i):
            iv = idx_v[pl.ds(i * L, L)]
            ob[pl.ds(i * L, L)] = plsc.load_gather(rb, [iv])

        out_cp[p] = pltpu.async_copy(ob, out_hbm.at[rbase + di], osems[p])
    out_cp[0].wait()
    out_cp[1].wait()


@jax.jit
def _sc_topk(fp2, scores):
    mesh = plsc.VectorSubcoreMesh(core_axis_name="c", subcore_axis_name="s")
    fn = functools.partial(
        pl.kernel,
        out_type=jax.ShapeDtypeStruct((B * D, KPAD), jnp.float32),
        mesh=mesh,
        compiler_params=pltpu.CompilerParams(
            needs_layout_passes=False, use_tc_tiling_on_sc=True),
        scratch_types=[
            pltpu.VMEM((N,), jnp.float32),
            pltpu.VMEM((N,), jnp.float32),
            pltpu.VMEM((L * HSTRIDE,), jnp.int32),
            pltpu.VMEM((256 + L,), jnp.int32),
            pltpu.VMEM((KPAD,), jnp.int32),
            pltpu.VMEM((KPAD,), jnp.float32),
            pltpu.VMEM((KPAD,), jnp.float32),
            pltpu.VMEM((NVEC + L,), jnp.int32),
            pltpu.VMEM((NVEC + L,), jnp.int32),
            pltpu.VMEM((NVEC + L,), jnp.int32),
            pltpu.VMEM((NVEC + L,), jnp.int32),
            pltpu.VMEM_SHARED((8, KPAD), jnp.int32),
            pltpu.SemaphoreType.DMA,
            pltpu.SemaphoreType.DMA,
            pltpu.SemaphoreType.DMA,
            pltpu.SemaphoreType.DMA,
        ],
    )(_topk_body)
    return fn(fp2, scores)


def kernel(frames, scores):
    fp2 = frames.transpose(0, 2, 1).reshape(B * D, N)
    out = _sc_topk(fp2, scores)
    return out[:, :K].reshape(B, D, K).transpose(0, 2, 1)
